# Initial kernel scaffold; baseline (speedup 1.0000x reference)
import jax
import jax.numpy as jnp
from jax.experimental import pallas as pl


def kernel(x):
    raise NotImplementedError("write your pallas kernel here")


# speedup vs baseline: 1.6928x; 1.6928x over previous
import jax
import jax.numpy as jnp
from jax.experimental import pallas as pl
from jax.experimental.pallas import tpu as pltpu

_LANES = 128


def _round_up(a: int, m: int) -> int:
    return (a + m - 1) // m * m


def _cdiv(a: int, m: int) -> int:
    return (a + m - 1) // m


def _pool_body(x_ref, max_ref, avg_ref, *, inv_len):
    x = x_ref[...].astype(jnp.float32)
    max_ref[...] = jnp.max(x, axis=1, keepdims=True).astype(max_ref.dtype)
    avg_ref[...] = (jnp.sum(x, axis=1, keepdims=True)
                    * inv_len).astype(avg_ref.dtype)


def _pool_body_chunked(x_ref, max_ref, avg_ref, *, n_chunks, chunk_l,
                       length, inv_len):
    acc_m = x_ref[:, : _LANES].astype(jnp.float32)
    acc_s = acc_m
    for q in range(1, n_chunks):
        lo = q * chunk_l
        xq = x_ref[:, lo:lo + chunk_l].astype(jnp.float32)
        if lo + chunk_l > length:
            col = lo + jax.lax.broadcasted_iota(jnp.int32, xq.shape, 1)
            valid = col < length
            acc_m = jnp.maximum(acc_m, jnp.where(valid, xq, -jnp.inf))
            acc_s = acc_s + jnp.where(valid, xq, 0.0)
        else:
            acc_m = jnp.maximum(acc_m, xq)
            acc_s = acc_s + xq
    max_ref[...] = jnp.max(acc_m, axis=1, keepdims=True).astype(max_ref.dtype)
    avg_ref[...] = (jnp.sum(acc_s, axis=1, keepdims=True)
                    * inv_len).astype(avg_ref.dtype)


def _concat_pool(x, *, target_block_bytes=8 * 1024 * 1024):
    N, C, L = x.shape
    NR = N * C
    x2 = x.reshape(NR, L)

    sub = {4: 8, 2: 16, 1: 32}.get(jnp.dtype(x.dtype).itemsize, 8)
    row_bytes = L * jnp.dtype(x.dtype).itemsize
    br = max(sub, _round_up(max(1, target_block_bytes // row_bytes), sub))
    if NR > sub:
        br = min(br, _round_up(_cdiv(NR, 2), sub))
    br = min(br, _round_up(NR, sub))
    nr_blocks = _cdiv(NR, br)

    out_shapes = (jax.ShapeDtypeStruct((NR, 1), x.dtype),
                  jax.ShapeDtypeStruct((NR, 1), x.dtype))
    out_specs = [pl.BlockSpec((br, 1), lambda i: (i, 0)),
                 pl.BlockSpec((br, 1), lambda i: (i, 0))]

    import functools
    if L <= 8192:
        body = functools.partial(_pool_body, inv_len=1.0 / L)
    else:
        chunk_l = _LANES
        body = functools.partial(
            _pool_body_chunked, n_chunks=_cdiv(L, chunk_l), chunk_l=chunk_l,
            length=L, inv_len=1.0 / L)

    mx2, av2 = pl.pallas_call(
        body,
        out_shape=out_shapes,
        grid=(nr_blocks,),
        in_specs=[pl.BlockSpec((br, L), lambda i: (i, 0))],
        out_specs=out_specs,
        compiler_params=pltpu.CompilerParams(
            dimension_semantics=("parallel",)),
    )(x2)

    mx = mx2.reshape(N, C)
    av = av2.reshape(N, C)
    return jnp.concatenate([mx, av], axis=1)[:, :, None]


def kernel(x):
    return _concat_pool(x)
